# Initial kernel scaffold; baseline (speedup 1.0000x reference)
import jax
import jax.numpy as jnp
from jax.experimental import pallas as pl


def kernel(x, table):
    raise NotImplementedError("write your pallas kernel here")


# speedup vs baseline: 1.4359x; 1.4359x over previous
import functools

import jax
import jax.numpy as jnp
from jax import lax
from jax.experimental import pallas as pl
from jax.experimental.pallas import tpu as pltpu
from jax.experimental.pallas import tpu_sc as plsc

D_MODEL = 32
_NC = 2
_NS = 16
_NW = _NC * _NS
_CB = 128


def _gather_body(table_hbm, idx_hbm, out_hbm, idx_v, rows_v, sem):
    wid = lax.axis_index("s") * _NC + lax.axis_index("c")
    nch = idx_hbm.shape[1]
    per_w = nch * _CB
    pltpu.sync_copy(idx_hbm.at[wid], idx_v)

    def step(j, carry):
        pltpu.async_copy(table_hbm.at[idx_v.at[j]], rows_v, sem).wait()
        pltpu.sync_copy(rows_v, out_hbm.at[pl.ds(wid * per_w + j * _CB, _CB)])
        return carry

    lax.fori_loop(0, nch, step, 0)


def kernel(x, table):
    rows, cols = x.shape
    b = rows * cols
    per_w = b // _NW
    nch = per_w // _CB
    idx = x.reshape(-1).astype(jnp.int32).reshape(_NW, nch, _CB)

    gather = functools.partial(
        pl.kernel,
        mesh=plsc.VectorSubcoreMesh(core_axis_name="c", subcore_axis_name="s"),
        out_type=jax.ShapeDtypeStruct((b, D_MODEL), jnp.float32),
        scratch_types=[
            pltpu.VMEM((nch, _CB), jnp.int32),
            pltpu.VMEM((_CB, D_MODEL), jnp.float32),
            pltpu.SemaphoreType.DMA,
        ],
        compiler_params=pltpu.CompilerParams(use_tc_tiling_on_sc=False),
    )(_gather_body)

    out = gather(table, idx)
    return out.reshape(rows, cols, D_MODEL)


# speedup vs baseline: 1.4430x; 1.0050x over previous
import jax
import jax.numpy as jnp
from jax import lax
from jax.experimental import pallas as pl
from jax.experimental.pallas import tpu as pltpu
from jax.experimental.pallas import tpu_sc as plsc

D = 32
_NC = 2
_NS = 16
_NW = _NC * _NS
_RPC = 4
_IPC = _RPC * 26
_CPW = 128


def _gather_body(table_hbm, idx_hbm, out_hbm, idx_v, rows_v,
                 sem_in0, sem_in1, sem_out0, sem_out1):
    wid = lax.axis_index("s") * _NC + lax.axis_index("c")
    row0 = wid * (_CPW * _RPC)
    sems_in = (sem_in0, sem_in1)
    sems_out = (sem_out0, sem_out1)
    pltpu.sync_copy(idx_hbm.at[wid], idx_v)

    def start_gather(k, b):
        pltpu.async_copy(table_hbm.at[idx_v.at[k]], rows_v.at[b], sems_in[b])

    def wait_gather(b):
        pltpu.make_async_copy(
            table_hbm.at[pl.ds(0, _IPC)], rows_v.at[b], sems_in[b]
        ).wait()

    def start_writes(k, b):
        for m in range(_RPC):
            pltpu.async_copy(
                rows_v.at[b].at[pl.ds(26 * m, 26)],
                out_hbm.at[row0 + k * _RPC + m],
                sems_out[b],
            )

    def drain_writes(b):
        for m in range(_RPC):
            pltpu.make_async_copy(
                rows_v.at[b].at[pl.ds(26 * m, 26)], out_hbm.at[0], sems_out[b]
            ).wait()

    start_gather(0, 0)

    def step(g, carry):
        for b in range(2):
            k = 2 * g + b
            wait_gather(b)
            start_writes(k, b)

            @pl.when(k + 1 < _CPW)
            def _():
                nb = 1 - b

                @pl.when(k >= 1)
                def _():
                    drain_writes(nb)

                start_gather(k + 1, nb)

        return carry

    lax.fori_loop(0, _CPW // 2, step, 0)
    drain_writes(0)
    drain_writes(1)


def kernel(x, table):
    rows, cols = x.shape
    idx = x.astype(jnp.int32).reshape(_NW, _CPW, _IPC)

    gather = pl.kernel(
        _gather_body,
        mesh=plsc.VectorSubcoreMesh(core_axis_name="c", subcore_axis_name="s"),
        out_type=jax.ShapeDtypeStruct((rows, cols, D), jnp.float32),
        scratch_types=[
            pltpu.VMEM((_CPW, _IPC), jnp.int32),
            pltpu.VMEM((2, _IPC, D), jnp.float32),
            pltpu.SemaphoreType.DMA,
            pltpu.SemaphoreType.DMA,
            pltpu.SemaphoreType.DMA,
            pltpu.SemaphoreType.DMA,
        ],
        compiler_params=pltpu.CompilerParams(use_tc_tiling_on_sc=False),
    )

    return gather(table, idx)
